# baseline (speedup 1.0000x reference)
import jax
import jax.numpy as jnp
from jax.experimental import pallas as pl
from jax.experimental.pallas import tpu as pltpu

_T = 2048
_E = 768
_DIM = 1024
_HEADS = 16
_DH = 32
_DIM_HEADS = _HEADS * _DH
_OUP = 1024
_TBLK = 512
_NBLK = _T // _TBLK
_LOG2E = 1.4426950408889634
_C = _DH ** -0.5 * _LOG2E
_AUG = _DH + 8


def _mono_kernel(x_ref, wt3_ref, wqk_ref, wv_ref, wo_ref, b_ref, out_ref,
                 lhs_s, rhs_s, va_s, o_s):
    step = pl.program_id(0)

    def proj(i):
        hT = jax.lax.dot_general(
            wt3_ref[...], x_ref[...], (((1,), (1,)), ((), ())),
            preferred_element_type=jnp.float32)
        qkT = jax.lax.dot_general(
            wqk_ref[...], hT, (((1,), (0,)), ((), ())),
            preferred_element_type=jnp.float32)
        vT = jax.lax.dot_general(
            wv_ref[...], hT, (((1,), (0,)), ((), ())),
            preferred_element_type=jnp.float32)
        hid = jax.lax.broadcasted_iota(jnp.int32, (_HEADS, _DIM_HEADS), 0)
        row = jax.lax.broadcasted_iota(jnp.int32, (_HEADS, _DIM_HEADS), 1)
        selT = (row // _DH == hid).astype(jnp.float32)
        nsqT = jax.lax.dot_general(
            selT, qkT * qkT, (((1,), (0,)), ((), ())),
            preferred_element_type=jnp.float32)
        normT = jnp.maximum(jnp.sqrt(nsqT), 1e-12)
        invbT = jax.lax.dot_general(
            selT, 1.0 / normT, (((0,), (0,)), ((), ())),
            preferred_element_type=jnp.float32)
        kT = qkT * invbT

        def split(z):
            return z.reshape(_HEADS, _DH, _TBLK)

        e0 = (jax.lax.broadcasted_iota(jnp.int32, (_HEADS, 8, _TBLK), 1) == 0)
        e0 = e0.astype(jnp.float32)
        mrow = jnp.broadcast_to((-_C * normT)[:, None, :], (_HEADS, 8, _TBLK))
        sl = slice(i * _TBLK, (i + 1) * _TBLK)
        lhs_s[:, :, sl] = jnp.concatenate(
            [split(_C * qkT), mrow * e0], axis=1).astype(jnp.bfloat16)
        rhs_s[:, :, sl] = jnp.concatenate(
            [split(kT), e0], axis=1).astype(jnp.bfloat16)
        va_s[:, :, sl] = jnp.concatenate(
            [split(vT), e0], axis=1).astype(jnp.bfloat16)

    for i in range(_NBLK):
        @pl.when(step == i)
        def _(i=i):
            proj(i)

    @pl.when((step >= _NBLK) & (step < _NBLK + _HEADS))
    def _attn():
        h = step - _NBLK
        lhs = lhs_s[h]
        rhs = rhs_s[h]
        va = va_s[h]
        numT = jnp.zeros((_AUG, _T), jnp.float32)
        jt = _T // 4
        for j in range(4):
            jsl = slice(j * jt, (j + 1) * jt)
            sT_t = jax.lax.dot_general(
                rhs[:, jsl], lhs, (((0,), (0,)), ((), ())),
                preferred_element_type=jnp.float32)
            eT_t = jnp.exp2(sT_t.astype(jnp.bfloat16))
            numT = numT + jax.lax.dot_general(
                va[:, jsl], eT_t, (((1,), (0,)), ((), ())),
                preferred_element_type=jnp.float32)
        denom = numT[_DH:_DH + 1, :] - 1.0
        o_s[h] = (numT[:_DH, :] - va[:_DH, :].astype(jnp.float32)) / denom

    for i in range(_NBLK):
        @pl.when(step == _NBLK + _HEADS + i)
        def _(i=i):
            oT = o_s[:, :, i * _TBLK:(i + 1) * _TBLK].reshape(
                _DIM_HEADS, _TBLK)
            out_ref[...] = jax.lax.dot_general(
                oT, wo_ref[...], (((0,), (1,)), ((), ())),
                preferred_element_type=jnp.float32) + b_ref[...]


def kernel(x, W_temp3, W_toqk, W_tov, W_out, b_out):
    x2 = x[0]
    nsteps = _NBLK + _HEADS + _NBLK
    out = pl.pallas_call(
        _mono_kernel,
        grid=(nsteps,),
        in_specs=[
            pl.BlockSpec((_TBLK, _E), lambda s: (jnp.minimum(s, _NBLK - 1), 0)),
            pl.BlockSpec((_DIM, _E), lambda s: (0, 0)),
            pl.BlockSpec((_DIM_HEADS, _DIM), lambda s: (0, 0)),
            pl.BlockSpec((_DIM_HEADS, _DIM), lambda s: (0, 0)),
            pl.BlockSpec((_OUP, _DIM_HEADS), lambda s: (0, 0)),
            pl.BlockSpec((1, _OUP), lambda s: (0, 0)),
        ],
        out_specs=pl.BlockSpec(
            (_TBLK, _OUP),
            lambda s: (jnp.clip(s - (_NBLK + _HEADS), 0, _NBLK - 1), 0)),
        out_shape=jax.ShapeDtypeStruct((_T, _OUP), jnp.float32),
        scratch_shapes=[
            pltpu.VMEM((_HEADS, _AUG, _T), jnp.bfloat16),
            pltpu.VMEM((_HEADS, _AUG, _T), jnp.bfloat16),
            pltpu.VMEM((_HEADS, _AUG, _T), jnp.bfloat16),
            pltpu.VMEM((_HEADS, _DH, _T), jnp.float32),
        ],
    )(x2, W_temp3, W_toqk, W_tov, W_out, b_out.reshape(1, _OUP))

    return out.reshape(1, _T, _OUP)


# speedup vs baseline: 1.0199x; 1.0199x over previous
import jax
import jax.numpy as jnp
from jax.experimental import pallas as pl
from jax.experimental.pallas import tpu as pltpu

_T = 2048
_E = 768
_DIM = 1024
_HEADS = 16
_DH = 32
_DIM_HEADS = _HEADS * _DH
_OUP = 1024
_TBLK = 512
_NBLK = _T // _TBLK
_PBLK = 1024
_NPROJ = _T // _PBLK
_LOG2E = 1.4426950408889634
_C = _DH ** -0.5 * _LOG2E
_AUG = _DH + 8


def _mono_kernel(x_ref, wt3_ref, wqk_ref, wv_ref, wo_ref, b_ref, out_ref,
                 lhs_s, rhs_s, va_s, o_s):
    step = pl.program_id(0)

    def proj(i):
        hT = jax.lax.dot_general(
            wt3_ref[...], x_ref[...], (((1,), (1,)), ((), ())),
            preferred_element_type=jnp.float32)
        qkT = jax.lax.dot_general(
            wqk_ref[...], hT, (((1,), (0,)), ((), ())),
            preferred_element_type=jnp.float32)
        vT = jax.lax.dot_general(
            wv_ref[...], hT, (((1,), (0,)), ((), ())),
            preferred_element_type=jnp.float32)
        hid = jax.lax.broadcasted_iota(jnp.int32, (_HEADS, _DIM_HEADS), 0)
        row = jax.lax.broadcasted_iota(jnp.int32, (_HEADS, _DIM_HEADS), 1)
        selT = (row // _DH == hid).astype(jnp.float32)
        nsqT = jax.lax.dot_general(
            selT, qkT * qkT, (((1,), (0,)), ((), ())),
            preferred_element_type=jnp.float32)
        normT = jnp.maximum(jnp.sqrt(nsqT), 1e-12)
        invbT = jax.lax.dot_general(
            selT, 1.0 / normT, (((0,), (0,)), ((), ())),
            preferred_element_type=jnp.float32)
        kT = qkT * invbT

        def split(z):
            return z.reshape(_HEADS, _DH, _PBLK)

        e0 = (jax.lax.broadcasted_iota(jnp.int32, (_HEADS, 8, _PBLK), 1) == 0)
        e0 = e0.astype(jnp.float32)
        mrow = jnp.broadcast_to((-_C * normT)[:, None, :], (_HEADS, 8, _PBLK))
        sl = slice(i * _PBLK, (i + 1) * _PBLK)
        lhs_s[:, :, sl] = jnp.concatenate(
            [split(_C * qkT), mrow * e0], axis=1).astype(jnp.bfloat16)
        rhs_s[:, :, sl] = jnp.concatenate(
            [split(kT), e0], axis=1).astype(jnp.bfloat16)
        va_s[:, :, sl] = jnp.concatenate(
            [split(vT), e0], axis=1).astype(jnp.bfloat16)

    for i in range(_NPROJ):
        @pl.when(step == i)
        def _(i=i):
            proj(i)

    @pl.when((step >= _NPROJ) & (step < _NPROJ + _HEADS))
    def _attn():
        h = step - _NPROJ
        lhs = lhs_s[h]
        rhs = rhs_s[h]
        va = va_s[h]
        sT = jax.lax.dot_general(
            rhs, lhs, (((0,), (0,)), ((), ())),
            preferred_element_type=jnp.float32)
        eT = jnp.exp2(sT.astype(jnp.bfloat16))
        numT = jax.lax.dot_general(
            va, eT, (((1,), (0,)), ((), ())),
            preferred_element_type=jnp.float32)
        denom = numT[_DH:_DH + 1, :] - 1.0
        o_s[h] = (numT[:_DH, :] - va[:_DH, :].astype(jnp.float32)) / denom

    for i in range(_NBLK):
        @pl.when(step == _NPROJ + _HEADS + i)
        def _(i=i):
            oT = o_s[:, :, i * _TBLK:(i + 1) * _TBLK].reshape(
                _DIM_HEADS, _TBLK)
            out_ref[...] = jax.lax.dot_general(
                oT, wo_ref[...], (((0,), (1,)), ((), ())),
                preferred_element_type=jnp.float32) + b_ref[...]


def kernel(x, W_temp3, W_toqk, W_tov, W_out, b_out):
    x2 = x[0]
    nsteps = _NPROJ + _HEADS + _NBLK
    out = pl.pallas_call(
        _mono_kernel,
        grid=(nsteps,),
        in_specs=[
            pl.BlockSpec((_PBLK, _E), lambda s: (jnp.minimum(s, _NPROJ - 1), 0)),
            pl.BlockSpec((_DIM, _E), lambda s: (0, 0)),
            pl.BlockSpec((_DIM_HEADS, _DIM), lambda s: (0, 0)),
            pl.BlockSpec((_DIM_HEADS, _DIM), lambda s: (0, 0)),
            pl.BlockSpec((_OUP, _DIM_HEADS), lambda s: (0, 0)),
            pl.BlockSpec((1, _OUP), lambda s: (0, 0)),
        ],
        out_specs=pl.BlockSpec(
            (_TBLK, _OUP),
            lambda s: (jnp.clip(s - (_NPROJ + _HEADS), 0, _NBLK - 1), 0)),
        out_shape=jax.ShapeDtypeStruct((_T, _OUP), jnp.float32),
        scratch_shapes=[
            pltpu.VMEM((_HEADS, _AUG, _T), jnp.bfloat16),
            pltpu.VMEM((_HEADS, _AUG, _T), jnp.bfloat16),
            pltpu.VMEM((_HEADS, _AUG, _T), jnp.bfloat16),
            pltpu.VMEM((_HEADS, _DH, _T), jnp.float32),
        ],
    )(x2, W_temp3, W_toqk, W_tov, W_out, b_out.reshape(1, _OUP))

    return out.reshape(1, _T, _OUP)
